# baseline (speedup 1.0000x reference)
import functools

import jax
import jax.numpy as jnp
from jax import lax
from jax.experimental import pallas as pl
from jax.experimental.pallas import tpu as pltpu
from jax.experimental.pallas import tpu_sc as plsc

MIN_POS = -1.2
MAX_POS = 0.6
MAX_SPEED = 0.07

N = 16384
TBL = 1024 * 1024
NC = 2
NS = 16
NW = NC * NS
CHUNK = N // NW
IDX_BLK = 128
NBLK = CHUNK // IDX_BLK

TC_GRID = 8
TC_BLK = N // TC_GRID

_B0 = float(-MIN_POS)
_B1 = float(MAX_SPEED)
_M0 = float(1023.999 / (MAX_POS - MIN_POS))
_M1 = float(1023.999 / (2.0 * MAX_SPEED))


def _tc_idx_body(inp_ref, idx_ref):
    x = inp_ref[...]
    col = lax.broadcasted_iota(jnp.int32, x.shape, 1)
    is0 = col == 0
    bv = jnp.where(is0, _B0, _B1).astype(jnp.float32)
    mv = jnp.where(is0, _M0, _M1).astype(jnp.float32)
    v = ((x + bv) * mv).astype(jnp.int32)
    t = jnp.where(
        is0,
        (v >> 3) * 8192 + (v & 7) * 128,
        (v >> 7) * 1024 + (v & 127),
    )
    idx_ref[...] = jnp.sum(t, axis=1)


def _tc_idx(inp):
    return pl.pallas_call(
        _tc_idx_body,
        grid=(TC_GRID,),
        in_specs=[pl.BlockSpec((TC_BLK, 2), lambda i: (i, 0))],
        out_specs=pl.BlockSpec((TC_BLK,), lambda i: (i,)),
        out_shape=jax.ShapeDtypeStruct((N,), jnp.int32),
    )(inp)


_mesh = plsc.VectorSubcoreMesh(core_axis_name="c", subcore_axis_name="s")


@functools.partial(
    pl.kernel,
    mesh=_mesh,
    out_type=jax.ShapeDtypeStruct((N,), jnp.float32),
    scratch_types=[
        pltpu.VMEM((1, CHUNK), jnp.int32),
        pltpu.VMEM((1, CHUNK), jnp.float32),
        pltpu.SemaphoreType.DMA,
    ],
    compiler_params=pltpu.CompilerParams(needs_layout_passes=False),
)
def _sc_gather(idx_hbm, data_hbm, out_hbm, idx_v, out_v, sem):
    wid = lax.axis_index("s") * NC + lax.axis_index("c")
    base = wid * CHUNK

    pltpu.sync_copy(idx_hbm.at[pl.ds(base, CHUNK)], idx_v.at[0])

    flat = data_hbm.at[0, pl.ds(0, IDX_BLK)]
    copies = [
        pltpu.async_copy(
            flat.at[idx_v.at[0, pl.ds(j * IDX_BLK, IDX_BLK)]],
            out_v.at[0, pl.ds(j * IDX_BLK, IDX_BLK)],
            sem,
        )
        for j in range(NBLK)
    ]
    for cp in copies:
        cp.wait()

    pltpu.sync_copy(out_v.at[0], out_hbm.at[pl.ds(base, CHUNK)])


def kernel(inp, data):
    return _sc_gather(_tc_idx(inp), data)


# speedup vs baseline: 1.8112x; 1.8112x over previous
import functools

import jax
import jax.numpy as jnp
from jax import lax
from jax.experimental import pallas as pl
from jax.experimental.pallas import tpu as pltpu
from jax.experimental.pallas import tpu_sc as plsc

MIN_POS = -1.2
MAX_POS = 0.6
MAX_SPEED = 0.07

N = 16384
NC = 2
NS = 16
NW = NC * NS
CHUNK = N // NW
LANES = 16
GROUPS = CHUNK // LANES
IDX_BLK = 128
NBLK = CHUNK // IDX_BLK

_B0 = float(-MIN_POS)
_B1 = float(MAX_SPEED)
_M0 = float(1023.999 / (MAX_POS - MIN_POS))
_M1 = float(1023.999 / (2.0 * MAX_SPEED))

_mesh = plsc.VectorSubcoreMesh(core_axis_name="c", subcore_axis_name="s")


@functools.partial(
    pl.kernel,
    mesh=_mesh,
    out_type=jax.ShapeDtypeStruct((N,), jnp.float32),
    scratch_types=[
        pltpu.VMEM((CHUNK,), jnp.float32),
        pltpu.VMEM((CHUNK,), jnp.float32),
        pltpu.VMEM((1, CHUNK), jnp.int32),
        pltpu.VMEM((1, CHUNK), jnp.float32),
        pltpu.SemaphoreType.DMA,
        pltpu.SemaphoreType.DMA,
    ],
)
def _sc_lookup(inp_t_hbm, data_hbm, out_hbm, pos_v, vel_v, idx_v, out_v,
               sem_in, sem_g):
    wid = lax.axis_index("s") * NC + lax.axis_index("c")
    base = wid * CHUNK

    cp_p = pltpu.async_copy(inp_t_hbm.at[0, pl.ds(base, CHUNK)], pos_v, sem_in)
    cp_v = pltpu.async_copy(inp_t_hbm.at[1, pl.ds(base, CHUNK)], vel_v, sem_in)
    cp_p.wait()
    cp_v.wait()

    b0 = jnp.float32(_B0)
    b1 = jnp.float32(_B1)
    m0 = jnp.float32(_M0)
    m1 = jnp.float32(_M1)

    flat = data_hbm.at[0, pl.ds(0, IDX_BLK)]

    copies = []
    for j in range(NBLK):
        for g in range(j * (GROUPS // NBLK), (j + 1) * (GROUPS // NBLK)):
            pos = pos_v[pl.ds(g * LANES, LANES)]
            vel = vel_v[pl.ds(g * LANES, LANES)]
            r = ((pos + b0) * m0).astype(jnp.int32)
            c = ((vel + b1) * m1).astype(jnp.int32)
            idx_v[0, pl.ds(g * LANES, LANES)] = (
                (r >> 3) * 8192 + (r & 7) * 128 + (c >> 7) * 1024 + (c & 127)
            )
        copies.append(
            pltpu.async_copy(
                flat.at[idx_v.at[0, pl.ds(j * IDX_BLK, IDX_BLK)]],
                out_v.at[0, pl.ds(j * IDX_BLK, IDX_BLK)],
                sem_g,
            )
        )
    for cp in copies:
        cp.wait()

    pltpu.sync_copy(out_v.at[0], out_hbm.at[pl.ds(base, CHUNK)])


def kernel(inp, data):
    return _sc_lookup(inp.T, data)


# speedup vs baseline: 1.8120x; 1.0005x over previous
import functools

import jax
import jax.numpy as jnp
from jax import lax
from jax.experimental import pallas as pl
from jax.experimental.pallas import tpu as pltpu
from jax.experimental.pallas import tpu_sc as plsc

MIN_POS = -1.2
MAX_POS = 0.6
MAX_SPEED = 0.07

N = 16384
NC = 2
NS = 16
NW = NC * NS
CHUNK = N // NW
LANES = 16
GROUPS = CHUNK // LANES
IDX_BLK = 128
NBLK = CHUNK // IDX_BLK

_B0 = float(-MIN_POS)
_B1 = float(MAX_SPEED)
_M0 = float(1023.999 / (MAX_POS - MIN_POS))
_M1 = float(1023.999 / (2.0 * MAX_SPEED))

_mesh = plsc.VectorSubcoreMesh(core_axis_name="c", subcore_axis_name="s")


@functools.partial(
    pl.kernel,
    mesh=_mesh,
    out_type=jax.ShapeDtypeStruct((N,), jnp.float32),
    scratch_types=[
        pltpu.VMEM((CHUNK,), jnp.float32),
        pltpu.VMEM((CHUNK,), jnp.float32),
        pltpu.VMEM((1, CHUNK), jnp.int32),
        pltpu.VMEM((1, CHUNK), jnp.float32),
        pltpu.SemaphoreType.DMA,
        pltpu.SemaphoreType.DMA,
    ],
)
def _sc_lookup(inp_t_hbm, data_hbm, out_hbm, pos_v, vel_v, idx_v, out_v,
               sem_in, sem_g):
    wid = lax.axis_index("s") * NC + lax.axis_index("c")
    base = wid * CHUNK

    cp_p = pltpu.async_copy(inp_t_hbm.at[0, pl.ds(base, CHUNK)], pos_v, sem_in)
    cp_v = pltpu.async_copy(inp_t_hbm.at[1, pl.ds(base, CHUNK)], vel_v, sem_in)
    cp_p.wait()
    cp_v.wait()

    b0 = jnp.float32(_B0)
    b1 = jnp.float32(_B1)
    m0 = jnp.float32(_M0)
    m1 = jnp.float32(_M1)

    flat = data_hbm.at[0, pl.ds(0, IDX_BLK)]

    def _group(g, carry):
        pos = pos_v[pl.ds(g * LANES, LANES)]
        vel = vel_v[pl.ds(g * LANES, LANES)]
        r = ((pos + b0) * m0).astype(jnp.int32)
        c = ((vel + b1) * m1).astype(jnp.int32)
        idx_v[0, pl.ds(g * LANES, LANES)] = (
            ((r >> 3) << 13) + ((r & 7) << 7) + ((c >> 7) << 10) + (c & 127)
        )
        return carry

    copies = []
    for j in range(NBLK):
        lax.fori_loop(
            j * (GROUPS // NBLK), (j + 1) * (GROUPS // NBLK), _group, 0,
            unroll=False,
        )
        copies.append(
            pltpu.async_copy(
                flat.at[idx_v.at[0, pl.ds(j * IDX_BLK, IDX_BLK)]],
                out_v.at[0, pl.ds(j * IDX_BLK, IDX_BLK)],
                sem_g,
            )
        )
    for cp in copies:
        cp.wait()

    pltpu.sync_copy(out_v.at[0], out_hbm.at[pl.ds(base, CHUNK)])


def kernel(inp, data):
    return _sc_lookup(inp.T, data)
